# baseline (speedup 1.0000x reference)
import functools

import jax
import jax.numpy as jnp
from jax import lax
from jax.experimental import pallas as pl
from jax.experimental.pallas import tpu as pltpu
from jax.experimental.pallas import tpu_sc as plsc

CHUNK = 80
NSLOT = 4
LANES = 16
ZROWS = 125


def _body(nat, n_chunks, d_core, n_cores, n_sub,
          edge_hbm, src_hbm, dst_hbm, out_hbm,
          acc, rows0, rows1, rows2, rows3, neg0, neg1, neg2, neg3,
          idx0, idx1, idx2, idx3, zbuf,
          sem_l0, sem_l1, sem_l2, sem_l3, sem_s0, sem_s1, sem_s2, sem_s3):
    c = lax.axis_index("c")
    s = lax.axis_index("s")
    rows_per_sub = nat // n_sub
    col0 = c * d_core

    rows_b = (rows0, rows1, rows2, rows3)
    neg_b = (neg0, neg1, neg2, neg3)
    idx_b = (idx0, idx1, idx2, idx3)
    sem_l = (sem_l0, sem_l1, sem_l2, sem_l3)
    sem_s = (sem_s0, sem_s1, sem_s2, sem_s3)

    cnt = n_chunks // n_sub
    start = s * cnt

    def load_args(gi, b):
        ch = start + gi
        return (
            (src_hbm.at[ch], idx_b[b].at[0]),
            (dst_hbm.at[ch], idx_b[b].at[1]),
            (edge_hbm.at[pl.ds(ch * CHUNK, CHUNK),
                         pl.ds(col0, d_core)], rows_b[b]),
        )

    def start_loads(gi, b):
        for src, dst in load_args(gi, b):
            pltpu.async_copy(src, dst, sem_l[b])

    def wait_loads(gi, b):
        for src, dst in load_args(gi, b):
            pltpu.make_async_copy(src, dst, sem_l[b]).wait()

    def drain_scatters(b):
        pltpu.make_async_copy(rows_b[b], acc.at[idx_b[b].at[0]], sem_s[b]).wait()
        pltpu.make_async_copy(neg_b[b], acc.at[idx_b[b].at[1]], sem_s[b]).wait()

    start_loads(0, 0)
    start_loads(1, 1)

    ncg = d_core // LANES

    def zero_row(i, _):
        for k in range(ncg):
            zbuf[i, pl.ds(k * LANES, LANES)] = jnp.zeros((LANES,), jnp.float32)
        return 0

    lax.fori_loop(0, ZROWS, zero_row, 0)
    for b in range(rows_per_sub // ZROWS):
        base = s * rows_per_sub + b * ZROWS
        pltpu.sync_copy(zbuf, acc.at[pl.ds(base, ZROWS)])
    plsc.subcore_barrier()

    def loop_body(go, _):
        for b in range(NSLOT):
            gi = go * NSLOT + b
            pb = (b + NSLOT - 2) % NSLOT

            @pl.when((gi >= 2) & (gi <= cnt + 1))
            def _():
                drain_scatters(pb)

            @pl.when(gi + 2 < cnt)
            def _():
                start_loads(gi + 2, pb)

            @pl.when(gi < cnt)
            def _():
                wait_loads(gi, b)
                pltpu.async_copy(
                    rows_b[b], acc.at[idx_b[b].at[0]], sem_s[b], add=True)

                def neg_rows(i, _):
                    for r in range(5):
                        for k in range(ncg):
                            sl = pl.ds(k * LANES, LANES)
                            neg_b[b][i * 5 + r, sl] = -rows_b[b][i * 5 + r, sl]
                    return 0

                lax.fori_loop(0, CHUNK // 5, neg_rows, 0)

                pltpu.async_copy(
                    neg_b[b], acc.at[idx_b[b].at[1]], sem_s[b], add=True)

        return 0

    lax.fori_loop(0, (cnt + 2 + NSLOT - 1) // NSLOT + 1, loop_body, 0)
    plsc.subcore_barrier()

    row0 = s * rows_per_sub
    pltpu.sync_copy(acc.at[pl.ds(row0, rows_per_sub)],
                    out_hbm.at[pl.ds(row0, rows_per_sub), pl.ds(col0, d_core)])


def kernel(edge_attr, edge_src, edge_dst, species):
    nat = species.shape[0]
    n_edges, d_feat = edge_attr.shape
    info = plsc.get_sparse_core_info()
    n_cores, n_sub = info.num_cores, info.num_subcores
    d_core = d_feat // n_cores
    n_chunks = n_edges // CHUNK

    src2d = edge_src.reshape(n_chunks, CHUNK)
    dst2d = edge_dst.reshape(n_chunks, CHUNK)

    mesh = plsc.VectorSubcoreMesh(core_axis_name="c", subcore_axis_name="s")
    body = functools.partial(_body, nat, n_chunks, d_core, n_cores, n_sub)
    k = pl.kernel(
        body,
        out_type=jax.ShapeDtypeStruct((nat, d_feat), jnp.float32),
        mesh=mesh,
        scratch_types=[
            pltpu.VMEM_SHARED((nat, d_core), jnp.float32),
            pltpu.VMEM((CHUNK, d_core), jnp.float32),
            pltpu.VMEM((CHUNK, d_core), jnp.float32),
            pltpu.VMEM((CHUNK, d_core), jnp.float32),
            pltpu.VMEM((CHUNK, d_core), jnp.float32),
            pltpu.VMEM((CHUNK, d_core), jnp.float32),
            pltpu.VMEM((CHUNK, d_core), jnp.float32),
            pltpu.VMEM((CHUNK, d_core), jnp.float32),
            pltpu.VMEM((CHUNK, d_core), jnp.float32),
            pltpu.VMEM((2, CHUNK), jnp.int32),
            pltpu.VMEM((2, CHUNK), jnp.int32),
            pltpu.VMEM((2, CHUNK), jnp.int32),
            pltpu.VMEM((2, CHUNK), jnp.int32),
            pltpu.VMEM((ZROWS, d_core), jnp.float32),
            pltpu.SemaphoreType.DMA,
            pltpu.SemaphoreType.DMA,
            pltpu.SemaphoreType.DMA,
            pltpu.SemaphoreType.DMA,
            pltpu.SemaphoreType.DMA,
            pltpu.SemaphoreType.DMA,
            pltpu.SemaphoreType.DMA,
            pltpu.SemaphoreType.DMA,
        ],
        compiler_params=pltpu.CompilerParams(use_tc_tiling_on_sc=False),
    )
    return k(edge_attr, src2d, dst2d)
